# Initial kernel scaffold; baseline (speedup 1.0000x reference)
import jax
import jax.numpy as jnp
from jax.experimental import pallas as pl


def kernel(position_ids, pe):
    raise NotImplementedError("write your pallas kernel here")


# speedup vs baseline: 4.3380x; 4.3380x over previous
import functools

import jax
import jax.numpy as jnp
from jax import lax
from jax.experimental import pallas as pl
from jax.experimental.pallas import tpu as pltpu
from jax.experimental.pallas import tpu_sc as plsc

N_POSITIONS = 512
N_EMBD = 128

_B = 4096 * 200
_NC = 2
_NS = 16
_NW = _NC * _NS
_BPW = _B // _NW
_CHUNK = 128
_NCHUNK = _BPW // _CHUNK

_mesh = plsc.VectorSubcoreMesh(core_axis_name="c", subcore_axis_name="s")


@functools.partial(
    pl.kernel,
    mesh=_mesh,
    out_type=jax.ShapeDtypeStruct((_B, N_EMBD), jnp.float32),
    scratch_types=[
        pltpu.VMEM((_CHUNK,), jnp.int32),
        pltpu.VMEM((_CHUNK, N_EMBD), jnp.float32),
        pltpu.SemaphoreType.DMA,
    ],
)
def _gather_kernel(ids_hbm, table_hbm, out_hbm, idx_v, rows_v, sem):
    wid = lax.axis_index("s") * _NC + lax.axis_index("c")
    base = wid * _BPW

    def body(j, carry):
        b = base + j * _CHUNK
        pltpu.sync_copy(ids_hbm.at[pl.ds(b, _CHUNK)], idx_v)
        pltpu.async_copy(table_hbm.at[idx_v], rows_v, sem).wait()
        pltpu.sync_copy(rows_v, out_hbm.at[pl.ds(b, _CHUNK)])
        return carry

    lax.fori_loop(0, _NCHUNK, body, 0)


def kernel(position_ids, pe):
    ids_flat = jnp.reshape(position_ids, (_B,))
    out = _gather_kernel(ids_flat, pe)
    return jnp.reshape(out, (*position_ids.shape, N_EMBD))


# speedup vs baseline: 4.6425x; 1.0702x over previous
import functools

import jax
import jax.numpy as jnp
from jax import lax
from jax.experimental import pallas as pl
from jax.experimental.pallas import tpu as pltpu
from jax.experimental.pallas import tpu_sc as plsc

N_POSITIONS = 512
N_EMBD = 128

_B = 4096 * 200
_NC = 2
_NS = 16
_NW = _NC * _NS
_BPW = _B // _NW
_CHUNK = 128
_NCHUNK = _BPW // _CHUNK
_NBUF = 4
_NGROUP = _NCHUNK // _NBUF

_mesh = plsc.VectorSubcoreMesh(core_axis_name="c", subcore_axis_name="s")


@functools.partial(
    pl.kernel,
    mesh=_mesh,
    out_type=jax.ShapeDtypeStruct((_B, N_EMBD), jnp.float32),
    scratch_types=[
        pltpu.VMEM((_NBUF, _CHUNK), jnp.int32),
        pltpu.VMEM((_NBUF, _CHUNK, N_EMBD), jnp.float32),
        pltpu.SemaphoreType.DMA((_NBUF,)),
        pltpu.SemaphoreType.DMA((_NBUF,)),
        pltpu.SemaphoreType.DMA((_NBUF,)),
    ],
)
def _gather_kernel(ids_hbm, table_hbm, out_hbm, idx_v, rows_v, isem, gsem, wsem):
    wid = lax.axis_index("s") * _NC + lax.axis_index("c")
    base = wid * _BPW

    def off(c):
        return base + c * _CHUNK

    def fire_idx(c, b):
        pltpu.async_copy(ids_hbm.at[pl.ds(off(c), _CHUNK)], idx_v.at[b], isem.at[b])

    def wait_idx(c, b):
        pltpu.make_async_copy(
            ids_hbm.at[pl.ds(off(c), _CHUNK)], idx_v.at[b], isem.at[b]).wait()

    def fire_gather(b):
        pltpu.async_copy(table_hbm.at[idx_v.at[b]], rows_v.at[b], gsem.at[b])

    def wait_gather(b):
        pltpu.make_async_copy(
            table_hbm.at[idx_v.at[b]], rows_v.at[b], gsem.at[b]).wait()

    def fire_write(c, b):
        pltpu.async_copy(rows_v.at[b], out_hbm.at[pl.ds(off(c), _CHUNK)], wsem.at[b])

    def wait_write(c, b):
        pltpu.make_async_copy(
            rows_v.at[b], out_hbm.at[pl.ds(off(c), _CHUNK)], wsem.at[b]).wait()

    for b in range(_NBUF):
        fire_idx(b, b)
    for b in range(_NBUF):
        wait_idx(b, b)
        fire_gather(b)

    def body(g, carry):
        c0 = g * _NBUF
        for b in range(_NBUF):
            wait_gather(b)
            fire_write(c0 + b, b)

        @pl.when(g + 1 < _NGROUP)
        def _():
            c1 = c0 + _NBUF
            for b in range(_NBUF):
                fire_idx(c1 + b, b)
            for b in range(_NBUF):
                wait_write(c0 + b, b)
                wait_idx(c1 + b, b)
                fire_gather(b)

        return carry

    lax.fori_loop(0, _NGROUP, body, 0)

    for b in range(_NBUF):
        wait_write((_NGROUP - 1) * _NBUF + b, b)


def kernel(position_ids, pe):
    ids_flat = jnp.reshape(position_ids, (_B,))
    out = _gather_kernel(ids_flat, pe)
    return jnp.reshape(out, (*position_ids.shape, N_EMBD))


# speedup vs baseline: 10.5812x; 2.2792x over previous
import functools

import jax
import jax.numpy as jnp
from jax import lax
from jax.experimental import pallas as pl
from jax.experimental.pallas import tpu as pltpu
from jax.experimental.pallas import tpu_sc as plsc

N_POSITIONS = 512
N_EMBD = 128

_B = 4096 * 200
_NC = 2
_NS = 16
_NW = _NC * _NS
_BPW = _B // _NW
_CHUNK = 128
_NCHUNK = _BPW // _CHUNK
_NBUF = 2
_NGROUP = _NCHUNK // _NBUF

_mesh = plsc.VectorSubcoreMesh(core_axis_name="c", subcore_axis_name="s")


@functools.partial(
    pl.kernel,
    mesh=_mesh,
    out_type=jax.ShapeDtypeStruct((_B, N_EMBD), jnp.float32),
    scratch_types=[
        pltpu.VMEM_SHARED((N_POSITIONS, N_EMBD), jnp.float32),
        pltpu.VMEM((_NBUF, _CHUNK), jnp.int32),
        pltpu.VMEM((_NBUF, _CHUNK, N_EMBD), jnp.float32),
        pltpu.SemaphoreType.DMA((_NBUF,)),
        pltpu.SemaphoreType.DMA((_NBUF,)),
        pltpu.SemaphoreType.DMA((_NBUF,)),
    ],
)
def _gather_kernel(ids_hbm, table_hbm, out_hbm, tab_v, idx_v, rows_v, isem, gsem, wsem):
    wid = lax.axis_index("s") * _NC + lax.axis_index("c")
    base = wid * _BPW

    def off(c):
        return base + c * _CHUNK

    def fire_idx(c, b):
        pltpu.async_copy(ids_hbm.at[pl.ds(off(c), _CHUNK)], idx_v.at[b], isem.at[b])

    def wait_idx(c, b):
        pltpu.make_async_copy(
            ids_hbm.at[pl.ds(off(c), _CHUNK)], idx_v.at[b], isem.at[b]).wait()

    def fire_gather(b):
        pltpu.async_copy(tab_v.at[idx_v.at[b]], rows_v.at[b], gsem.at[b])

    def wait_gather(b):
        pltpu.make_async_copy(
            tab_v.at[idx_v.at[b]], rows_v.at[b], gsem.at[b]).wait()

    def fire_write(c, b):
        pltpu.async_copy(rows_v.at[b], out_hbm.at[pl.ds(off(c), _CHUNK)], wsem.at[b])

    def wait_write(c, b):
        pltpu.make_async_copy(
            rows_v.at[b], out_hbm.at[pl.ds(off(c), _CHUNK)], wsem.at[b]).wait()

    @pl.when(lax.axis_index("s") == 0)
    def _():
        pltpu.sync_copy(table_hbm, tab_v)

    plsc.subcore_barrier()

    for b in range(_NBUF):
        fire_idx(b, b)
    for b in range(_NBUF):
        wait_idx(b, b)
        fire_gather(b)

    def body(g, carry):
        c0 = g * _NBUF
        for b in range(_NBUF):
            wait_gather(b)
            fire_write(c0 + b, b)

        @pl.when(g + 1 < _NGROUP)
        def _():
            c1 = c0 + _NBUF
            for b in range(_NBUF):
                fire_idx(c1 + b, b)
            for b in range(_NBUF):
                wait_write(c0 + b, b)
                wait_idx(c1 + b, b)
                fire_gather(b)

        return carry

    lax.fori_loop(0, _NGROUP, body, 0)

    for b in range(_NBUF):
        wait_write((_NGROUP - 1) * _NBUF + b, b)


def kernel(position_ids, pe):
    ids_flat = jnp.reshape(position_ids, (_B,))
    out = _gather_kernel(ids_flat, pe)
    return jnp.reshape(out, (*position_ids.shape, N_EMBD))


# speedup vs baseline: 14.5799x; 1.3779x over previous
import functools

import jax
import jax.numpy as jnp
from jax import lax
from jax.experimental import pallas as pl
from jax.experimental.pallas import tpu as pltpu
from jax.experimental.pallas import tpu_sc as plsc

N_POSITIONS = 512
N_EMBD = 128

_B = 4096 * 200
_NC = 2
_NS = 16
_NW = _NC * _NS
_BPW = _B // _NW
_CHUNK = 128
_NCHUNK = _BPW // _CHUNK
_NBUF = 4
_NGROUP = _NCHUNK // _NBUF

_mesh = plsc.VectorSubcoreMesh(core_axis_name="c", subcore_axis_name="s")


@functools.partial(
    pl.kernel,
    mesh=_mesh,
    out_type=jax.ShapeDtypeStruct((_B, N_EMBD), jnp.float32),
    scratch_types=[
        pltpu.VMEM_SHARED((N_POSITIONS, N_EMBD), jnp.float32),
        pltpu.VMEM((_NBUF, _CHUNK), jnp.int32),
        pltpu.VMEM((_NBUF, _CHUNK, N_EMBD), jnp.float32),
        pltpu.SemaphoreType.DMA((_NBUF,)),
        pltpu.SemaphoreType.DMA((_NBUF,)),
        pltpu.SemaphoreType.DMA((_NBUF,)),
    ],
)
def _gather_kernel(ids_hbm, table_hbm, out_hbm, tab_v, idx_v, rows_v, isem, gsem, wsem):
    wid = lax.axis_index("s") * _NC + lax.axis_index("c")
    base = wid * _BPW

    def off(c):
        return base + c * _CHUNK

    def fire_idx(c, b):
        pltpu.async_copy(ids_hbm.at[pl.ds(off(c), _CHUNK)], idx_v.at[b], isem.at[b])

    def wait_idx(c, b):
        pltpu.make_async_copy(
            ids_hbm.at[pl.ds(off(c), _CHUNK)], idx_v.at[b], isem.at[b]).wait()

    def fire_gather(b):
        pltpu.async_copy(tab_v.at[idx_v.at[b]], rows_v.at[b], gsem.at[b])

    def wait_gather(b):
        pltpu.make_async_copy(
            tab_v.at[idx_v.at[b]], rows_v.at[b], gsem.at[b]).wait()

    def fire_write(c, b):
        pltpu.async_copy(rows_v.at[b], out_hbm.at[pl.ds(off(c), _CHUNK)], wsem.at[b])

    def wait_write(c, b):
        pltpu.make_async_copy(
            rows_v.at[b], out_hbm.at[pl.ds(off(c), _CHUNK)], wsem.at[b]).wait()

    @pl.when(lax.axis_index("s") == 0)
    def _():
        pltpu.sync_copy(table_hbm, tab_v)

    plsc.subcore_barrier()

    for b in range(_NBUF):
        fire_idx(b, b)
    for b in range(_NBUF):
        wait_idx(b, b)
        fire_gather(b)

    def body(g, carry):
        c0 = g * _NBUF
        for b in range(_NBUF):
            wait_gather(b)
            fire_write(c0 + b, b)

        @pl.when(g + 1 < _NGROUP)
        def _():
            c1 = c0 + _NBUF
            for b in range(_NBUF):
                fire_idx(c1 + b, b)
            for b in range(_NBUF):
                wait_write(c0 + b, b)
                wait_idx(c1 + b, b)
                fire_gather(b)

        return carry

    lax.fori_loop(0, _NGROUP, body, 0)

    for b in range(_NBUF):
        wait_write((_NGROUP - 1) * _NBUF + b, b)


def kernel(position_ids, pe):
    ids_flat = jnp.reshape(position_ids, (_B,))
    out = _gather_kernel(ids_flat, pe)
    return jnp.reshape(out, (*position_ids.shape, N_EMBD))


# speedup vs baseline: 14.6826x; 1.0070x over previous
import functools

import jax
import jax.numpy as jnp
from jax import lax
from jax.experimental import pallas as pl
from jax.experimental.pallas import tpu as pltpu
from jax.experimental.pallas import tpu_sc as plsc

N_POSITIONS = 512
N_EMBD = 128

_B = 4096 * 200
_NC = 2
_NS = 16
_NW = _NC * _NS
_BPW = _B // _NW
_CHUNK = 128
_NCHUNK = _BPW // _CHUNK
_NBUF = 5
_NGROUP = _NCHUNK // _NBUF

_mesh = plsc.VectorSubcoreMesh(core_axis_name="c", subcore_axis_name="s")


@functools.partial(
    pl.kernel,
    mesh=_mesh,
    out_type=jax.ShapeDtypeStruct((_B, N_EMBD), jnp.float32),
    scratch_types=[
        pltpu.VMEM_SHARED((N_POSITIONS, N_EMBD), jnp.float32),
        pltpu.VMEM((_NBUF, _CHUNK), jnp.int32),
        pltpu.VMEM((_NBUF, _CHUNK, N_EMBD), jnp.float32),
        pltpu.SemaphoreType.DMA((_NBUF,)),
        pltpu.SemaphoreType.DMA((_NBUF,)),
        pltpu.SemaphoreType.DMA((_NBUF,)),
    ],
)
def _gather_kernel(ids_hbm, table_hbm, out_hbm, tab_v, idx_v, rows_v, isem, gsem, wsem):
    wid = lax.axis_index("s") * _NC + lax.axis_index("c")
    base = wid * _BPW

    def off(c):
        return base + c * _CHUNK

    def fire_idx(c, b):
        pltpu.async_copy(ids_hbm.at[pl.ds(off(c), _CHUNK)], idx_v.at[b], isem.at[b])

    def wait_idx(c, b):
        pltpu.make_async_copy(
            ids_hbm.at[pl.ds(off(c), _CHUNK)], idx_v.at[b], isem.at[b]).wait()

    def fire_gather(b):
        pltpu.async_copy(tab_v.at[idx_v.at[b]], rows_v.at[b], gsem.at[b])

    def wait_gather(b):
        pltpu.make_async_copy(
            tab_v.at[idx_v.at[b]], rows_v.at[b], gsem.at[b]).wait()

    def fire_write(c, b):
        pltpu.async_copy(rows_v.at[b], out_hbm.at[pl.ds(off(c), _CHUNK)], wsem.at[b])

    def wait_write(c, b):
        pltpu.make_async_copy(
            rows_v.at[b], out_hbm.at[pl.ds(off(c), _CHUNK)], wsem.at[b]).wait()

    @pl.when(lax.axis_index("s") == 0)
    def _():
        pltpu.sync_copy(table_hbm, tab_v)

    plsc.subcore_barrier()

    for b in range(_NBUF):
        fire_idx(b, b)
    for b in range(_NBUF):
        wait_idx(b, b)
        fire_gather(b)

    def body(g, carry):
        c0 = g * _NBUF
        for b in range(_NBUF):
            wait_gather(b)
            fire_write(c0 + b, b)

        @pl.when(g + 1 < _NGROUP)
        def _():
            c1 = c0 + _NBUF
            for b in range(_NBUF):
                fire_idx(c1 + b, b)
            for b in range(_NBUF):
                wait_write(c0 + b, b)
                wait_idx(c1 + b, b)
                fire_gather(b)

        return carry

    lax.fori_loop(0, _NGROUP, body, 0)

    for b in range(_NBUF):
        wait_write((_NGROUP - 1) * _NBUF + b, b)


def kernel(position_ids, pe):
    ids_flat = jnp.reshape(position_ids, (_B,))
    out = _gather_kernel(ids_flat, pe)
    return jnp.reshape(out, (*position_ids.shape, N_EMBD))
